# Initial kernel scaffold; baseline (speedup 1.0000x reference)
import jax
import jax.numpy as jnp
from jax.experimental import pallas as pl


def kernel(x, W1, b1, W2, b2):
    raise NotImplementedError("write your pallas kernel here")


# speedup vs baseline: 1.0924x; 1.0924x over previous
import functools

import jax
import jax.numpy as jnp
from jax import lax
from jax.experimental import pallas as pl
from jax.experimental.pallas import tpu as pltpu
from jax.experimental.pallas import tpu_sc as plsc

B, S, DIM = 4, 8192, 768
HID = DIM // 4
K = S // 2
L = 16


def _scores_body(x_ref, w1_ref, b1_ref, w2_ref, b2_ref, out_ref):
    sb = x_ref.shape[1]
    xb = x_ref[...].reshape(B * sb, DIM)
    h = lax.dot_general(xb, w1_ref[...], (((1,), (1,)), ((), ())),
                        preferred_element_type=jnp.float32)
    h = jnp.maximum(h + b1_ref[...], 0.0)
    sc = lax.dot_general(h.astype(jnp.bfloat16),
                         w2_ref[...].reshape(HID, 1).astype(jnp.bfloat16),
                         (((1,), (0,)), ((), ())),
                         preferred_element_type=jnp.float32)
    out_ref[...] = sc.reshape(B, sb) + b2_ref[0, 0]


def _scores_tc(x, W1, b1, W2, b2):
    SB = 512
    return pl.pallas_call(
        _scores_body,
        grid=(S // SB,),
        in_specs=[
            pl.BlockSpec((B, SB, DIM), lambda j: (0, j, 0)),
            pl.BlockSpec((HID, DIM), lambda j: (0, 0)),
            pl.BlockSpec((1, HID), lambda j: (0, 0)),
            pl.BlockSpec((1, HID), lambda j: (0, 0)),
            pl.BlockSpec((1, 1), lambda j: (0, 0)),
        ],
        out_specs=pl.BlockSpec((B, SB), lambda j: (0, j)),
        out_shape=jax.ShapeDtypeStruct((B, S), jnp.float32),
    )(x, W1, b1.reshape(1, HID), W2.reshape(1, HID), b2.reshape(1, 1))


def _radix_pass(iota, hist, src_k, dst_k, src_p, dst_p, shift, nbits, n):
    chunk = n // L
    ndig = 1 << nbits
    dmask = ndig - 1
    lane_base = iota * chunk
    shift_v = jnp.full((L,), shift, jnp.int32)
    ones = jnp.ones((L,), jnp.int32)

    def zero_body(h, c):
        hist[pl.ds(h * L, L)] = jnp.zeros((L,), jnp.int32)
        return c
    lax.fori_loop(0, ndig, zero_body, 0)

    def hist_body(v, c):
        key = plsc.load_gather(src_k, [lane_base + v])
        d = lax.shift_right_logical(key, shift_v) & dmask
        plsc.addupdate_scatter(hist, [d * L + iota], ones)
        return c
    lax.fori_loop(0, chunk, hist_body, 0)

    def scan_body(h, carry):
        sl = pl.ds(h * L, L)
        cnt = hist[sl]
        hist[sl] = plsc.cumsum(cnt) - cnt + carry
        return carry + jnp.sum(cnt)
    lax.fori_loop(0, ndig, scan_body, jnp.int32(0))

    def perm_body(v, c):
        idx = lane_base + v
        key = plsc.load_gather(src_k, [idx])
        d = lax.shift_right_logical(key, shift_v) & dmask
        hidx = d * L + iota
        pos = plsc.load_gather(hist, [hidx])
        plsc.store_scatter(dst_k, [pos], key)
        if src_p is not None:
            plsc.store_scatter(dst_p, [pos], plsc.load_gather(src_p, [idx]))
        plsc.store_scatter(hist, [hidx], pos + 1)
        return c
    lax.fori_loop(0, chunk, perm_body, 0)


def _router_body(scores_hbm, mask_hbm, rout_hbm,
                 s_v, ka, kb, ia, ib, hist, e_v, mask_v, rout_v):
    cid = lax.axis_index("c")
    sid = lax.axis_index("s")
    row = sid * 2 + cid

    @pl.when(row < B)
    def _():
        pltpu.sync_copy(scores_hbm.at[row], s_v)
        iota = lax.iota(jnp.int32, L)

        def key_body(v, c):
            sl = pl.ds(v * L, L)
            i = lax.bitcast_convert_type(s_v[sl], jnp.int32)
            ka[sl] = jnp.where(i < 0, i, ~i & 0x7FFFFFFF)
            ia[sl] = iota + v * L
            return c
        lax.fori_loop(0, S // L, key_body, 0)

        _radix_pass(iota, hist, ka, kb, ia, ib, 0, 8, S)
        _radix_pass(iota, hist, kb, ka, ib, ia, 8, 8, S)
        _radix_pass(iota, hist, ka, kb, ia, ib, 16, 8, S)
        _radix_pass(iota, hist, kb, ka, ib, ia, 24, 8, S)

        k0 = ka[pl.ds(0, L)]
        vmax = jnp.max(lax.bitcast_convert_type(
            jnp.where(k0 < 0, k0, ~k0 & 0x7FFFFFFF), jnp.float32))

        def exp_body(r, zacc):
            sl = pl.ds(r * L, L)
            kk = ka[sl]
            f = lax.bitcast_convert_type(
                jnp.where(kk < 0, kk, ~kk & 0x7FFFFFFF), jnp.float32)
            e = jnp.exp(f - vmax)
            e_v[sl] = e
            return zacc + e
        zacc = lax.fori_loop(0, K // L, exp_body, jnp.zeros((L,), jnp.float32))
        zvec = jnp.broadcast_to(jnp.sum(zacc), (L,))
        zinv = jnp.ones((L,), jnp.float32) / zvec

        _radix_pass(iota, hist, ia, kb, None, None, 0, 8, K)
        _radix_pass(iota, hist, kb, ib, None, None, 8, 5, K)

        ones_i = jnp.ones((L,), jnp.int32)
        zeros_i = jnp.zeros((L,), jnp.int32)
        zeros_f = jnp.zeros((L,), jnp.float32)

        def sel_body(r, c):
            sl = pl.ds(r * L, L)
            pvec = ib[sl]
            plsc.store_scatter(rout_v, [pvec], e_v[sl] * zinv)
            plsc.store_scatter(mask_v, [pvec], ones_i)
            return c
        lax.fori_loop(0, K // L, sel_body, 0)

        def uns_body(r, c):
            sl = pl.ds(r * L, L)
            uvec = ia[sl]
            plsc.store_scatter(rout_v, [uvec], zeros_f)
            plsc.store_scatter(mask_v, [uvec], zeros_i)
            return c
        lax.fori_loop(K // L, S // L, uns_body, 0)

        pltpu.sync_copy(mask_v, mask_hbm.at[row])
        pltpu.sync_copy(rout_v, rout_hbm.at[row])


def _router_sc(scores):
    mesh = plsc.VectorSubcoreMesh(core_axis_name="c", subcore_axis_name="s")
    fn = pl.kernel(
        _router_body,
        out_type=(jax.ShapeDtypeStruct((B, S), jnp.int32),
                  jax.ShapeDtypeStruct((B, S), jnp.float32)),
        mesh=mesh,
        compiler_params=pltpu.CompilerParams(needs_layout_passes=False),
        scratch_types=[
            pltpu.VMEM((S,), jnp.float32),
            pltpu.VMEM((S,), jnp.int32),
            pltpu.VMEM((S,), jnp.int32),
            pltpu.VMEM((S,), jnp.int32),
            pltpu.VMEM((S,), jnp.int32),
            pltpu.VMEM((256 * L,), jnp.int32),
            pltpu.VMEM((K,), jnp.float32),
            pltpu.VMEM((S,), jnp.int32),
            pltpu.VMEM((S,), jnp.float32),
        ],
    )
    return fn(scores)


def kernel(x, W1, b1, W2, b2):
    scores = _scores_tc(x, W1, b1, W2, b2)
    mask_i, routing = _router_sc(scores)
    return mask_i.astype(bool), routing
